# Initial kernel scaffold; baseline (speedup 1.0000x reference)
import jax
import jax.numpy as jnp
from jax.experimental import pallas as pl


def kernel(x, edge_index, W1, W2):
    raise NotImplementedError("write your pallas kernel here")


# speedup vs baseline: 10.4071x; 10.4071x over previous
import functools

import jax
import jax.numpy as jnp
from jax import lax
from jax.experimental import pallas as pl
from jax.experimental.pallas import tpu as pltpu
from jax.experimental.pallas import tpu_sc as plsc

_N = 10000
_NPAD = 10240
_E = 320000
_D = 128
_CHUNK = 128
_NTILES = 32
_CPT = 79
_EPT = _CHUNK * _CPT
_EPAD = _NTILES * _EPT
_RPT = _NPAD // 16
_DEGW = 16

_mesh = plsc.VectorSubcoreMesh(core_axis_name="c", subcore_axis_name="s")


@functools.partial(
    pl.kernel,
    out_type=jax.ShapeDtypeStruct((2 * _NPAD, _DEGW), jnp.float32),
    mesh=_mesh,
    scratch_types=[
        pltpu.VMEM((_CHUNK,), jnp.int32),
        pltpu.VMEM((_CHUNK, _DEGW), jnp.float32),
        pltpu.VMEM((_CHUNK, _DEGW), jnp.float32),
        pltpu.SemaphoreType.DMA,
        pltpu.VMEM_SHARED((_NPAD, _DEGW), jnp.float32),
    ],
)
def _sc_deg(dst_hbm, zeros_hbm, ones_hbm, out_hbm, dst_v, ones_v, buf_v, sem,
            shared_deg):
    c = lax.axis_index("c")
    s = lax.axis_index("s")
    wid = c * 16 + s
    pltpu.sync_copy(zeros_hbm, buf_v)
    pltpu.sync_copy(ones_hbm, ones_v)
    for j in range(_RPT // _CHUNK):
        pltpu.sync_copy(buf_v, shared_deg.at[pl.ds(s * _RPT + j * _CHUNK, _CHUNK)])
    plsc.subcore_barrier()

    def body(i, carry):
        base = pl.multiple_of(wid * _EPT + i * _CHUNK, 8)
        pltpu.sync_copy(dst_hbm.at[pl.ds(base, _CHUNK)], dst_v)
        pltpu.sync_copy(ones_v, shared_deg.at[dst_v], add=True)
        return carry

    lax.fori_loop(0, _CPT, body, 0)
    plsc.subcore_barrier()
    for j in range(_RPT // _CHUNK):
        r0 = s * _RPT + j * _CHUNK
        pltpu.sync_copy(shared_deg.at[pl.ds(r0, _CHUNK)], buf_v)
        pltpu.sync_copy(buf_v, out_hbm.at[pl.ds(c * _NPAD + r0, _CHUNK)])


@functools.partial(
    pl.kernel,
    out_type=jax.ShapeDtypeStruct((2 * _NPAD, _D), jnp.float32),
    mesh=_mesh,
    scratch_types=[
        pltpu.VMEM((_CHUNK,), jnp.int32),
        pltpu.VMEM((_CHUNK,), jnp.int32),
        pltpu.VMEM((_CHUNK, _D), jnp.float32),
        pltpu.SemaphoreType.DMA,
        pltpu.VMEM_SHARED((_NPAD, _D), jnp.float32),
    ],
)
def _sc_spmm(h_hbm, src_hbm, dst_hbm, zeros_hbm, out_hbm, src_v, dst_v,
             rows_v, sem, shared_acc):
    c = lax.axis_index("c")
    s = lax.axis_index("s")
    wid = c * 16 + s
    pltpu.sync_copy(zeros_hbm, rows_v)
    for j in range(_RPT // _CHUNK):
        pltpu.sync_copy(rows_v, shared_acc.at[pl.ds(s * _RPT + j * _CHUNK, _CHUNK)])
    plsc.subcore_barrier()

    def body(i, carry):
        base = pl.multiple_of(wid * _EPT + i * _CHUNK, 8)
        pltpu.sync_copy(src_hbm.at[pl.ds(base, _CHUNK)], src_v)
        pltpu.sync_copy(dst_hbm.at[pl.ds(base, _CHUNK)], dst_v)
        pltpu.async_copy(h_hbm.at[src_v], rows_v, sem).wait()
        pltpu.sync_copy(rows_v, shared_acc.at[dst_v], add=True)
        return carry

    lax.fori_loop(0, _CPT, body, 0)
    plsc.subcore_barrier()
    for j in range(_RPT // _CHUNK):
        r0 = s * _RPT + j * _CHUNK
        pltpu.sync_copy(shared_acc.at[pl.ds(r0, _CHUNK)], rows_v)
        pltpu.sync_copy(rows_v, out_hbm.at[pl.ds(c * _NPAD + r0, _CHUNK)])


_BR = 1024


def _dinv_of(d0, d1):
    return lax.rsqrt(d0[:, 0:1] + d1[:, 0:1] + 1.0)


def _mm1_body(x_ref, w_ref, d0_ref, d1_ref, o_ref):
    dinv = _dinv_of(d0_ref[...], d1_ref[...])
    o_ref[...] = dinv * jnp.dot(x_ref[...], w_ref[...],
                                preferred_element_type=jnp.float32)


def _mm2_body(p0_ref, p1_ref, h1s_ref, d0_ref, d1_ref, w_ref, o_ref):
    dinv = _dinv_of(d0_ref[...], d1_ref[...])
    agg = dinv * (p0_ref[...] + p1_ref[...] + h1s_ref[...])
    h = jnp.maximum(agg, 0.0)
    o_ref[...] = dinv * jnp.dot(h, w_ref[...],
                                preferred_element_type=jnp.float32)


def _final_body(q0_ref, q1_ref, h2s_ref, d0_ref, d1_ref, o_ref):
    dinv = _dinv_of(d0_ref[...], d1_ref[...])
    agg = dinv * (q0_ref[...] + q1_ref[...] + h2s_ref[...])
    m = jnp.max(agg, axis=-1, keepdims=True)
    z = agg - m
    lse = jnp.log(jnp.sum(jnp.exp(z), axis=-1, keepdims=True))
    o_ref[...] = z - lse


def _row_spec(w):
    return pl.BlockSpec((_BR, w), lambda i: (i, 0))


def _full_spec(r, c):
    return pl.BlockSpec((r, c), lambda i: (0, 0))


def _mm1(x_pad, W1, d0, d1):
    return pl.pallas_call(
        _mm1_body,
        grid=(_NPAD // _BR,),
        in_specs=[_row_spec(_D), _full_spec(_D, _D), _row_spec(_DEGW),
                  _row_spec(_DEGW)],
        out_specs=_row_spec(_D),
        out_shape=jax.ShapeDtypeStruct((_NPAD, _D), jnp.float32),
    )(x_pad, W1, d0, d1)


def _mm2(p0, p1, h1s, d0, d1, W2):
    return pl.pallas_call(
        _mm2_body,
        grid=(_NPAD // _BR,),
        in_specs=[_row_spec(_D), _row_spec(_D), _row_spec(_D),
                  _row_spec(_DEGW), _row_spec(_DEGW), _full_spec(_D, _D)],
        out_specs=_row_spec(_D),
        out_shape=jax.ShapeDtypeStruct((_NPAD, _D), jnp.float32),
    )(p0, p1, h1s, d0, d1, W2)


def _final(q0, q1, h2s, d0, d1):
    return pl.pallas_call(
        _final_body,
        grid=(_NPAD // _BR,),
        in_specs=[_row_spec(_D), _row_spec(_D), _row_spec(_D),
                  _row_spec(_DEGW), _row_spec(_DEGW)],
        out_specs=_row_spec(_D),
        out_shape=jax.ShapeDtypeStruct((_NPAD, _D), jnp.float32),
    )(q0, q1, h2s, d0, d1)


def kernel(x, edge_index, W1, W2):
    src = edge_index[0]
    dst = edge_index[1]
    npadE = _EPAD - _E
    srcp = jnp.concatenate([src, jnp.zeros((npadE,), jnp.int32)])
    dstp = jnp.concatenate([dst, jnp.full((npadE,), _NPAD - 1, jnp.int32)])
    x_pad = jnp.pad(x, ((0, _NPAD - _N), (0, 0)))
    zeros16 = jnp.zeros((_CHUNK, _DEGW), jnp.float32)
    ones16 = jnp.ones((_CHUNK, _DEGW), jnp.float32)
    zeros128 = jnp.zeros((_CHUNK, _D), jnp.float32)

    deg = _sc_deg(dstp, zeros16, ones16)
    d0 = deg[:_NPAD]
    d1 = deg[_NPAD:]

    h1s = _mm1(x_pad, W1, d0, d1)
    P = _sc_spmm(h1s, srcp, dstp, zeros128)
    h2s = _mm2(P[:_NPAD], P[_NPAD:], h1s, d0, d1, W2)
    Q = _sc_spmm(h2s, srcp, dstp, zeros128)
    out = _final(Q[:_NPAD], Q[_NPAD:], h2s, d0, d1)
    return out[:_N]
